# Initial kernel scaffold; baseline (speedup 1.0000x reference)
import jax
import jax.numpy as jnp
from jax.experimental import pallas as pl


def kernel(x, evectors, batch, weight, bias, ev_scales):
    raise NotImplementedError("write your pallas kernel here")


# speedup vs baseline: 3.7432x; 3.7432x over previous
import functools

import jax
import jax.numpy as jnp
from jax.experimental import pallas as pl
from jax.experimental.pallas import tpu as pltpu

N = 262144
H = 256
E = 32
G = 1024
EPS = 1e-5

B = 2048
B1 = 2048
NB = N // B
NBH = NB // 2
SMAX = G + NB
SW = 32

_F32 = jnp.float32
_BF16 = jnp.bfloat16


def _split_hi_lo(a):
    hi = a.astype(_BF16)
    lo = (a - hi.astype(_F32)).astype(_BF16)
    return hi, lo


def _dotg(a, b, dims):
    return jax.lax.dot_general(a, b, dimension_numbers=dims,
                               preferred_element_type=_F32)


def _dot3(a, b, dims):
    ah, al = _split_hi_lo(a)
    bh, bl = _split_hi_lo(b)
    return _dotg(ah, bh, dims) + (_dotg(ah, bl, dims) + _dotg(al, bh, dims))


def _contrib_kernel(ev_ref, x_ref, acc_ref):
    i = pl.program_id(1)

    @pl.when(i == 0)
    def _():
        acc_ref[...] = jnp.zeros_like(acc_ref)

    dims = (((0,), (0,)), ((), ()))
    c = _dot3(ev_ref[...], x_ref[...], dims)
    acc_ref[...] += c[None]


def _seg_onehot_t(sstart, sbase, base_row, rows_1b):
    sid = jnp.zeros(rows_1b.shape, jnp.int32)
    for j in range(SW + 1):
        stj = sstart[jnp.minimum(sbase + j, SMAX - 1)] - base_row
        sid = sid + jnp.where(rows_1b >= stj, 1, 0)
    sid_b = jnp.broadcast_to(sid - 1, (SW,) + rows_1b.shape[1:])
    iota_sub = jax.lax.broadcasted_iota(jnp.int32, sid_b.shape, 0)
    return jnp.where(sid_b == iota_sub, 1.0, 0.0).astype(_BF16)


def _stats_kernel(sstart, send, sg, sptr, x_ref, ev_ref, scales_ref,
                  contrib_ref, sq_ref, cnt_ref, sqloc_ref):
    c = pl.program_id(0)
    i = pl.program_id(1)
    b = c * NBH + i

    @pl.when(i == 0)
    def _():
        sq_ref[...] = jnp.zeros_like(sq_ref)
        cnt_ref[...] = jnp.zeros_like(cnt_ref)

    sc = (1.0 + scales_ref[...]) * (contrib_ref[0] + contrib_ref[1])
    dims = (((1,), (0,)), ((), ()))
    mean = _dot3(ev_ref[...], sc, dims)
    out = x_ref[...] - mean
    sqb = (out * out).astype(_BF16)

    base_row = b * B
    s0 = sptr[b]
    s1 = sptr[b + 1]
    nseg = s1 - s0
    rounds = jax.lax.div(nseg + (SW - 1), SW)
    rows_1b = jax.lax.broadcasted_iota(jnp.int32, (1, B), 1)

    def round_body(r, _):
        sbase = s0 + r * SW
        oh_t = _seg_onehot_t(sstart, sbase, base_row, rows_1b)
        sqloc_ref[...] = _dotg(oh_t, sqb, (((1,), (0,)), ((), ())))
        rem = jnp.minimum(nseg - r * SW, SW)

        def seg_body(jj, _):
            sidx = sbase + jj
            g = sg[sidx]
            cntv = (send[sidx] - sstart[sidx]).astype(_F32)
            sq_ref[0, pl.ds(g, 1), :] += sqloc_ref[pl.ds(jj, 1), :]
            cnt_ref[0, pl.ds(g, 1), :] += jnp.full((1, 128), 1.0, _F32) * cntv
            return 0

        jax.lax.fori_loop(0, rem, seg_body, 0)
        return 0

    jax.lax.fori_loop(0, rounds, round_body, 0)


def _finalize_kernel(sqp_ref, cntp_ref, contrib_ref, scales_ref, w_ref,
                     winv_ref, sch_ref, scl_ref):
    cnt128 = jnp.maximum(cntp_ref[0] + cntp_ref[1], 1.0)
    cnt = jnp.concatenate([cnt128, cnt128], axis=1)
    var = (sqp_ref[0] + sqp_ref[1]) / cnt
    winv_ref[...] = w_ref[...] * jax.lax.rsqrt(var + EPS)
    sc = (1.0 + scales_ref[...]) * (contrib_ref[0] + contrib_ref[1])
    sch = sc.astype(_BF16)
    sch_ref[...] = sch
    scl_ref[...] = (sc - sch.astype(_F32)).astype(_BF16)


def _norm_kernel(sstart, send, sg, sptr, x_ref, ev_ref, sch_ref, scl_ref,
                 winv_ref, bias_ref, y_ref, wloc_ref, rs_ref):
    c = pl.program_id(0)
    i = pl.program_id(1)
    b = c * NBH + i

    evh, evl = _split_hi_lo(ev_ref[...])
    dims = (((1,), (0,)), ((), ()))
    mean = _dotg(evh, sch_ref[...], dims) + (
        _dotg(evh, scl_ref[...], dims) + _dotg(evl, sch_ref[...], dims))
    out = x_ref[...] - mean

    base_row = b * B
    s0 = sptr[b]
    s1 = sptr[b + 1]
    nseg = s1 - s0
    rounds = jax.lax.div(nseg + (SW - 1), SW)
    rows_1b = jax.lax.broadcasted_iota(jnp.int32, (1, B), 1)
    tdims = (((0,), (0,)), ((), ()))

    def gather_rows(sbase, rem):
        wloc_ref[...] = jnp.zeros_like(wloc_ref)

        def seg_body(jj, _):
            g = sg[sbase + jj]
            wloc_ref[pl.ds(jj, 1), :] = winv_ref[pl.ds(g, 1), :]
            return 0
        jax.lax.fori_loop(0, rem, seg_body, 0)

    def round_dot(sbase):
        oh_t = _seg_onehot_t(sstart, sbase, base_row, rows_1b)
        wl = wloc_ref[...]
        wlh, wll = _split_hi_lo(wl)
        return _dotg(oh_t, wlh, tdims) + _dotg(oh_t, wll, tdims)

    gather_rows(s0, jnp.minimum(nseg, SW))
    rs_ref[...] = round_dot(s0)

    def round_body(r, _):
        sbase = s0 + r * SW
        gather_rows(sbase, jnp.minimum(nseg - r * SW, SW))
        rs_ref[...] += round_dot(sbase)
        return 0

    jax.lax.fori_loop(1, rounds, round_body, 0)
    y_ref[...] = out * rs_ref[...] + bias_ref[...]


def kernel(x, evectors, batch, weight, bias, ev_scales):
    bi = batch.astype(jnp.int32)

    first = jnp.concatenate(
        [jnp.ones((1,), jnp.bool_), bi[1:] != bi[:-1]])
    first = first | ((jnp.arange(N, dtype=jnp.int32) % B) == 0)
    seg_start = jnp.nonzero(first, size=SMAX, fill_value=N)[0].astype(jnp.int32)
    nxt = jnp.concatenate([seg_start[1:], jnp.full((1,), N, jnp.int32)])
    blk_end = (seg_start // B + 1) * B
    seg_end = jnp.minimum(nxt, blk_end)
    seg_g = bi[jnp.minimum(seg_start, N - 1)]
    seg_ptr = jnp.searchsorted(
        seg_start, jnp.arange(NB + 1, dtype=jnp.int32) * B,
        side='left').astype(jnp.int32)

    w2 = weight.reshape(1, H)
    b2 = bias.reshape(1, H)

    nb1h = (N // B1) // 2
    contribp = pl.pallas_call(
        _contrib_kernel,
        out_shape=jax.ShapeDtypeStruct((2, E, H), _F32),
        grid=(2, nb1h),
        in_specs=[
            pl.BlockSpec((B1, E), lambda c, i: (c * nb1h + i, 0)),
            pl.BlockSpec((B1, H), lambda c, i: (c * nb1h + i, 0)),
        ],
        out_specs=pl.BlockSpec((1, E, H), lambda c, i: (c, 0, 0)),
        compiler_params=pltpu.CompilerParams(
            dimension_semantics=("parallel", "arbitrary")),
        name="gn2_contrib",
    )(evectors, x)

    sqp, cntp = pl.pallas_call(
        _stats_kernel,
        out_shape=(
            jax.ShapeDtypeStruct((2, G, H), _F32),
            jax.ShapeDtypeStruct((2, G, 128), _F32),
        ),
        grid_spec=pltpu.PrefetchScalarGridSpec(
            num_scalar_prefetch=4,
            grid=(2, NBH),
            in_specs=[
                pl.BlockSpec((B, H), lambda c, i, *_: (c * NBH + i, 0)),
                pl.BlockSpec((B, E), lambda c, i, *_: (c * NBH + i, 0)),
                pl.BlockSpec((E, H), lambda c, i, *_: (0, 0)),
                pl.BlockSpec((2, E, H), lambda c, i, *_: (0, 0, 0)),
            ],
            out_specs=(
                pl.BlockSpec((1, G, H), lambda c, i, *_: (c, 0, 0)),
                pl.BlockSpec((1, G, 128), lambda c, i, *_: (c, 0, 0)),
            ),
            scratch_shapes=[pltpu.VMEM((SW, H), _F32)],
        ),
        compiler_params=pltpu.CompilerParams(
            dimension_semantics=("parallel", "arbitrary")),
        name="gn2_stats",
    )(seg_start, seg_end, seg_g, seg_ptr, x, evectors, ev_scales, contribp)

    winv, sch, scl = pl.pallas_call(
        _finalize_kernel,
        out_shape=(
            jax.ShapeDtypeStruct((G, H), _F32),
            jax.ShapeDtypeStruct((E, H), _BF16),
            jax.ShapeDtypeStruct((E, H), _BF16),
        ),
        name="gn2_finalize",
    )(sqp, cntp, contribp, ev_scales, w2)

    y = pl.pallas_call(
        _norm_kernel,
        out_shape=jax.ShapeDtypeStruct((N, H), _F32),
        grid_spec=pltpu.PrefetchScalarGridSpec(
            num_scalar_prefetch=4,
            grid=(2, NBH),
            in_specs=[
                pl.BlockSpec((B, H), lambda c, i, *_: (c * NBH + i, 0)),
                pl.BlockSpec((B, E), lambda c, i, *_: (c * NBH + i, 0)),
                pl.BlockSpec((E, H), lambda c, i, *_: (0, 0)),
                pl.BlockSpec((E, H), lambda c, i, *_: (0, 0)),
                pl.BlockSpec((G, H), lambda c, i, *_: (0, 0)),
                pl.BlockSpec((1, H), lambda c, i, *_: (0, 0)),
            ],
            out_specs=pl.BlockSpec((B, H), lambda c, i, *_: (c * NBH + i, 0)),
            scratch_shapes=[
                pltpu.VMEM((SW, H), _F32),
                pltpu.VMEM((B, H), _F32),
            ],
        ),
        compiler_params=pltpu.CompilerParams(
            dimension_semantics=("parallel", "arbitrary")),
        name="gn2_norm",
    )(seg_start, seg_end, seg_g, seg_ptr, x, evectors, sch, scl, winv, b2)

    return y
